# baseline (speedup 1.0000x reference)
import functools

import numpy as np
import jax
import jax.numpy as jnp
from jax import lax
from jax.experimental import pallas as pl
from jax.experimental.pallas import tpu as pltpu
from jax.experimental.pallas import tpu_sc as plsc

N = 10000
NP = 10240
E = 320000
D = 128
NTY = 16
ALPHA = 10.0

NC = 2
NS = 16
L = 16
NW = NC * NS
G = 80
CH = 4032
EPAD = CH * G
CPT = CH // NW
CPS = CH // NS
NB = 21
NBG = NB * G
NBL_C = CPT // NB
NBL_A = CPS // NB
RPT = NP // NS
RCH = 80
NRCH = RPT // RCH

_PAD = EPAD - E
_PAD_SRC = np.asarray((np.arange(_PAD) * 97) % N, dtype=np.int32)
_PAD_DST = np.asarray(N + (np.arange(_PAD) % (NP - N)), dtype=np.int32)
_PAD_EF = np.zeros((_PAD,), dtype=np.int32)


def _sc_body(feat_hbm, src_hbm, dst_hbm, ef_hbm, ewt_hbm,
             hpart_hbm,
             ewt_v, sA, dA, eA,
             dstc0, dstc1, dstc2, cva0, cva1, cva2,
             rows, nbuf,
             h_sh, deg_sh,
             semg0, semg1, semg2,
             sems0, sems1, sems2):
    c = lax.axis_index("c")
    s = lax.axis_index("s")
    wid = s * NC + c
    zero = jnp.zeros((L,), jnp.float32)
    semg = [semg0, semg1, semg2]
    sems = [sems0, sems1, sems2]
    cva = [cva0, cva1, cva2]
    dstc = [dstc0, dstc1, dstc2]
    rows0 = rows.at[0]

    def copy_dst_idx(b, j):
        for i in range(G // L):
            dstc[b][pl.ds(i * L, L)] = dA[pl.ds(j * G + i * L, L)]

    @pl.loop(0, RCH)
    def _zero_rows0(r):
        for j in range(D // L):
            rows0[r, pl.ds(j * L, L)] = zero

    @pl.loop(0, RPT // L)
    def _zero_nbuf(i):
        nbuf[pl.ds(i * L, L)] = zero

    pltpu.sync_copy(nbuf.at[pl.ds(0, RPT)], deg_sh.at[pl.ds(s * RPT, RPT)])
    for k in range(NRCH):
        pltpu.sync_copy(rows0, h_sh.at[pl.ds(s * RPT + k * RCH, RCH)])
    pltpu.sync_copy(ewt_hbm, ewt_v)
    plsc.subcore_barrier()

    for blk in range(NBL_A):
        base = (s * CPS + blk * NB) * G
        pltpu.sync_copy(dst_hbm.at[pl.ds(base, NBG)], dA)
        pltpu.sync_copy(ef_hbm.at[pl.ds(base, NBG)], eA)

        @pl.loop(0, NB // 3)
        def _deg(k):
            for b in range(3):
                j = 3 * k + b

                @pl.when(k >= 1)
                def _wait_prev():
                    pltpu.make_async_copy(
                        cva[b], deg_sh.at[dstc[b]], sems[b]).wait()

                copy_dst_idx(b, j)
                for i in range(G // L):
                    t = eA[pl.ds(j * G + i * L, L)]
                    cva[b][pl.ds(i * L, L)] = plsc.load_gather(ewt_v, [t])
                pltpu.async_copy(cva[b], deg_sh.at[dstc[b]], sems[b],
                                 add=True)

        for b in range(3):
            pltpu.make_async_copy(
                cva[b], deg_sh.at[dstc[b]], sems[b]).wait()

    plsc.subcore_barrier()

    pltpu.sync_copy(deg_sh, nbuf)

    @pl.loop(0, NP // L)
    def _norm(i):
        d = nbuf[pl.ds(i * L, L)]
        nbuf[pl.ds(i * L, L)] = 1.0 / jnp.maximum(d, 1.0)

    cv = cva0
    for blk in range(NBL_C):
        base = (wid * CPT + blk * NB) * G
        pltpu.sync_copy(src_hbm.at[pl.ds(base, NBG)], sA)
        pltpu.sync_copy(dst_hbm.at[pl.ds(base, NBG)], dA)
        pltpu.sync_copy(ef_hbm.at[pl.ds(base, NBG)], eA)
        for j in range(2):
            pltpu.async_copy(feat_hbm.at[sA.at[pl.ds(j * G, G // 2)]],
                             rows.at[j].at[pl.ds(0, G // 2)], semg[j])
            pltpu.async_copy(feat_hbm.at[sA.at[pl.ds(j * G + G // 2, G // 2)]],
                             rows.at[j].at[pl.ds(G // 2, G // 2)], semg[j])

        @pl.loop(0, NB // 3)
        def _msg(k):
            for b in range(3):
                j = 3 * k + b
                rb = rows.at[b]
                for i in range(G // L):
                    t = eA[pl.ds(j * G + i * L, L)]
                    ew = plsc.load_gather(ewt_v, [t])
                    sidx = sA[pl.ds(j * G + i * L, L)]
                    nm = plsc.load_gather(nbuf, [sidx])
                    cv[pl.ds(i * L, L)] = ew * nm
                copy_dst_idx(b, j)
                pltpu.make_async_copy(
                    feat_hbm.at[sA.at[pl.ds(j * G, G // 2)]],
                    rb.at[pl.ds(0, G // 2)], semg[b]).wait()
                pltpu.make_async_copy(
                    feat_hbm.at[sA.at[pl.ds(j * G + G // 2, G // 2)]],
                    rb.at[pl.ds(G // 2, G // 2)], semg[b]).wait()

                @pl.loop(0, G, unroll=2)
                def _scale(r):
                    cb = plsc.load_gather(cv, [jnp.full((L,), r, jnp.int32)])
                    for jj in range(D // L):
                        rb[r, pl.ds(jj * L, L)] = rb[r, pl.ds(jj * L, L)] * cb

                pltpu.async_copy(rb, h_sh.at[dstc[b]], sems[b], add=True)

                bn = (b + 2) % 3
                if b == 0:
                    @pl.when(k >= 1)
                    def _wait_s():
                        pltpu.make_async_copy(
                            rows.at[bn], h_sh.at[dstc[bn]], sems[bn]).wait()

                    pltpu.async_copy(
                        feat_hbm.at[sA.at[pl.ds((j + 2) * G, G // 2)]],
                        rows.at[bn].at[pl.ds(0, G // 2)], semg[bn])
                    pltpu.async_copy(
                        feat_hbm.at[sA.at[pl.ds((j + 2) * G + G // 2, G // 2)]],
                        rows.at[bn].at[pl.ds(G // 2, G // 2)], semg[bn])
                else:
                    @pl.when(k < NB // 3 - 1)
                    def _wait_issue():
                        pltpu.make_async_copy(
                            rows.at[bn], h_sh.at[dstc[bn]], sems[bn]).wait()
                        pltpu.async_copy(
                            feat_hbm.at[sA.at[pl.ds((j + 2) * G, G // 2)]],
                            rows.at[bn].at[pl.ds(0, G // 2)], semg[bn])
                        pltpu.async_copy(
                            feat_hbm.at[sA.at[pl.ds((j + 2) * G + G // 2, G // 2)]],
                            rows.at[bn].at[pl.ds(G // 2, G // 2)], semg[bn])

        for b in range(3):
            pltpu.make_async_copy(
                rows.at[b], h_sh.at[dstc[b]], sems[b]).wait()

    plsc.subcore_barrier()

    for k in range(NRCH):
        r0 = s * RPT + k * RCH
        pltpu.sync_copy(h_sh.at[pl.ds(r0, RCH)], rows0)
        pltpu.sync_copy(rows0, hpart_hbm.at[pl.ds(c * NP + r0, RCH)])


_sc_scatter = pl.kernel(
    _sc_body,
    out_type=jax.ShapeDtypeStruct((2 * NP, D), jnp.float32),
    mesh=plsc.VectorSubcoreMesh(core_axis_name="c", subcore_axis_name="s"),
    scratch_types=[
        pltpu.VMEM((NTY,), jnp.float32),
        pltpu.VMEM((NBG,), jnp.int32),
        pltpu.VMEM((NBG,), jnp.int32),
        pltpu.VMEM((NBG,), jnp.int32),
        pltpu.VMEM((G,), jnp.int32),
        pltpu.VMEM((G,), jnp.int32),
        pltpu.VMEM((G,), jnp.int32),
        pltpu.VMEM((G,), jnp.float32),
        pltpu.VMEM((G,), jnp.float32),
        pltpu.VMEM((G,), jnp.float32),
        pltpu.VMEM((3, G, D), jnp.float32),
        pltpu.VMEM((NP,), jnp.float32),
        pltpu.VMEM_SHARED((NP, D), jnp.float32),
        pltpu.VMEM_SHARED((NP,), jnp.float32),
        pltpu.SemaphoreType.DMA,
        pltpu.SemaphoreType.DMA,
        pltpu.SemaphoreType.DMA,
        pltpu.SemaphoreType.DMA,
        pltpu.SemaphoreType.DMA,
        pltpu.SemaphoreType.DMA,
    ],
    compiler_params=pltpu.CompilerParams(needs_layout_passes=False,
                                         use_tc_tiling_on_sc=False),
)


BR = 80


def _mm_body(f_ref, h0_ref, h1_ref, w_ref, b_ref, o_ref):
    x = f_ref[...] + h0_ref[...] + h1_ref[...]
    o_ref[...] = (
        jnp.dot(x, w_ref[...], preferred_element_type=jnp.float32) + b_ref[...]
    )


def _tc_matmul(feat, hpart, weight, bias2d):
    return pl.pallas_call(
        _mm_body,
        grid=(N // BR,),
        in_specs=[
            pl.BlockSpec((BR, D), lambda i: (i, 0)),
            pl.BlockSpec((BR, D), lambda i: (i, 0)),
            pl.BlockSpec((BR, D), lambda i: (i + NP // BR, 0)),
            pl.BlockSpec((D, D), lambda i: (0, 0)),
            pl.BlockSpec((1, D), lambda i: (0, 0)),
        ],
        out_specs=pl.BlockSpec((BR, D), lambda i: (i, 0)),
        out_shape=jax.ShapeDtypeStruct((N, D), jnp.float32),
    )(feat, hpart, hpart, weight, bias2d)


def kernel(feat, edge_index, e_feat, weight, bias, edge_weight):
    src = edge_index[0].astype(jnp.int32)
    dst = edge_index[1].astype(jnp.int32)
    ef = e_feat.astype(jnp.int32)
    srcp = jnp.concatenate([src, jnp.asarray(_PAD_SRC)])
    dstp = jnp.concatenate([dst, jnp.asarray(_PAD_DST)])
    efp = jnp.concatenate([ef, jnp.asarray(_PAD_EF)])
    ewt = jax.nn.leaky_relu(edge_weight * ALPHA, negative_slope=0.01)[:, 0]
    ewt2 = jnp.roll(ewt, 1)
    hpart = _sc_scatter(feat, srcp, dstp, efp, ewt2)
    return _tc_matmul(feat, hpart, weight, bias.reshape(1, D))


# speedup vs baseline: 1.0565x; 1.0565x over previous
import functools

import numpy as np
import jax
import jax.numpy as jnp
from jax import lax
from jax.experimental import pallas as pl
from jax.experimental.pallas import tpu as pltpu
from jax.experimental.pallas import tpu_sc as plsc

N = 10000
NP = 10240
E = 320000
D = 128
NTY = 16
ALPHA = 10.0

NC = 2
NS = 16
L = 16
NW = NC * NS
G = 80
CH = 4032
EPAD = CH * G
CPT = CH // NW
CPS = CH // NS
NB = 21
NBG = NB * G
NBL_C = CPT // NB
NBL_A = CPS // NB
RPT = NP // NS
RCH = 80
NRCH = RPT // RCH

_PAD = EPAD - E
_PAD_SRC = np.asarray((np.arange(_PAD) * 97) % N, dtype=np.int32)
_PAD_DST = np.asarray(N + (np.arange(_PAD) % (NP - N)), dtype=np.int32)
_PAD_EF = np.zeros((_PAD,), dtype=np.int32)


def _sc_body(feat_hbm, src_hbm, dst_hbm, ef_hbm, ewt_hbm,
             hpart_hbm,
             ewt_v, sA, dA, eA,
             dstc0, dstc1, dstc2, cva0, cva1, cva2,
             rows, nbuf,
             h_sh, deg_sh,
             semg0, semg1, semg2,
             sems0, sems1, sems2):
    c = lax.axis_index("c")
    s = lax.axis_index("s")
    wid = s * NC + c
    zero = jnp.zeros((L,), jnp.float32)
    semg = [semg0, semg1, semg2]
    sems = [sems0, sems1, sems2]
    cva = [cva0, cva1, cva2]
    dstc = [dstc0, dstc1, dstc2]
    rows0 = rows.at[0]

    def copy_dst_idx(b, j):
        for i in range(G // L):
            dstc[b][pl.ds(i * L, L)] = dA[pl.ds(j * G + i * L, L)]

    @pl.loop(0, RCH)
    def _zero_rows0(r):
        for j in range(D // L):
            rows0[r, pl.ds(j * L, L)] = zero

    @pl.loop(0, RPT // L)
    def _zero_nbuf(i):
        nbuf[pl.ds(i * L, L)] = zero

    pltpu.async_copy(nbuf.at[pl.ds(0, RPT)], deg_sh.at[pl.ds(s * RPT, RPT)],
                     sems[1])
    for k in range(NRCH):
        pltpu.async_copy(rows0, h_sh.at[pl.ds(s * RPT + k * RCH, RCH)],
                         sems[0])
    pltpu.async_copy(ewt_hbm, ewt_v, sems[2])
    for k in range(NRCH):
        pltpu.make_async_copy(rows0, h_sh.at[pl.ds(s * RPT + k * RCH, RCH)],
                              sems[0]).wait()
    pltpu.make_async_copy(nbuf.at[pl.ds(0, RPT)],
                          deg_sh.at[pl.ds(s * RPT, RPT)], sems[1]).wait()
    pltpu.make_async_copy(ewt_hbm, ewt_v, sems[2]).wait()
    plsc.subcore_barrier()

    for blk in range(NBL_A):
        base = (s * CPS + blk * NB) * G
        pltpu.async_copy(dst_hbm.at[pl.ds(base, NBG)], dA, semg[0])
        pltpu.async_copy(ef_hbm.at[pl.ds(base, NBG)], eA, semg[1])
        pltpu.make_async_copy(dst_hbm.at[pl.ds(base, NBG)], dA,
                              semg[0]).wait()
        pltpu.make_async_copy(ef_hbm.at[pl.ds(base, NBG)], eA,
                              semg[1]).wait()

        @pl.loop(0, NB // 3)
        def _deg(k):
            for b in range(3):
                j = 3 * k + b

                @pl.when(k >= 1)
                def _wait_prev():
                    pltpu.make_async_copy(
                        cva[b], deg_sh.at[dstc[b]], sems[b]).wait()

                copy_dst_idx(b, j)
                for i in range(G // L):
                    t = eA[pl.ds(j * G + i * L, L)]
                    cva[b][pl.ds(i * L, L)] = plsc.load_gather(ewt_v, [t])
                pltpu.async_copy(cva[b], deg_sh.at[dstc[b]], sems[b],
                                 add=True)

        for b in range(3):
            pltpu.make_async_copy(
                cva[b], deg_sh.at[dstc[b]], sems[b]).wait()

    plsc.subcore_barrier()

    pltpu.sync_copy(deg_sh, nbuf)

    @pl.loop(0, NP // L)
    def _norm(i):
        d = nbuf[pl.ds(i * L, L)]
        nbuf[pl.ds(i * L, L)] = 1.0 / jnp.maximum(d, 1.0)

    cv = cva0
    for blk in range(NBL_C):
        base = (wid * CPT + blk * NB) * G
        pltpu.async_copy(src_hbm.at[pl.ds(base, NBG)], sA, semg[0])
        pltpu.async_copy(dst_hbm.at[pl.ds(base, NBG)], dA, semg[1])
        pltpu.async_copy(ef_hbm.at[pl.ds(base, NBG)], eA, semg[2])
        pltpu.make_async_copy(src_hbm.at[pl.ds(base, NBG)], sA,
                              semg[0]).wait()
        pltpu.make_async_copy(dst_hbm.at[pl.ds(base, NBG)], dA,
                              semg[1]).wait()
        pltpu.make_async_copy(ef_hbm.at[pl.ds(base, NBG)], eA,
                              semg[2]).wait()
        for j in range(2):
            pltpu.async_copy(feat_hbm.at[sA.at[pl.ds(j * G, G // 2)]],
                             rows.at[j].at[pl.ds(0, G // 2)], semg[j])
            pltpu.async_copy(feat_hbm.at[sA.at[pl.ds(j * G + G // 2, G // 2)]],
                             rows.at[j].at[pl.ds(G // 2, G // 2)], semg[j])

        @pl.loop(0, NB // 3)
        def _msg(k):
            for b in range(3):
                j = 3 * k + b
                rb = rows.at[b]
                for i in range(G // L):
                    t = eA[pl.ds(j * G + i * L, L)]
                    ew = plsc.load_gather(ewt_v, [t])
                    sidx = sA[pl.ds(j * G + i * L, L)]
                    nm = plsc.load_gather(nbuf, [sidx])
                    cv[pl.ds(i * L, L)] = ew * nm
                copy_dst_idx(b, j)
                pltpu.make_async_copy(
                    feat_hbm.at[sA.at[pl.ds(j * G, G // 2)]],
                    rb.at[pl.ds(0, G // 2)], semg[b]).wait()
                pltpu.make_async_copy(
                    feat_hbm.at[sA.at[pl.ds(j * G + G // 2, G // 2)]],
                    rb.at[pl.ds(G // 2, G // 2)], semg[b]).wait()

                @pl.loop(0, G, unroll=4)
                def _scale(r):
                    cb = plsc.load_gather(cv, [jnp.full((L,), r, jnp.int32)])
                    for jj in range(D // L):
                        rb[r, pl.ds(jj * L, L)] = rb[r, pl.ds(jj * L, L)] * cb

                pltpu.async_copy(rb, h_sh.at[dstc[b]], sems[b], add=True)

                bn = (b + 2) % 3
                if b == 0:
                    @pl.when(k >= 1)
                    def _wait_s():
                        pltpu.make_async_copy(
                            rows.at[bn], h_sh.at[dstc[bn]], sems[bn]).wait()

                    pltpu.async_copy(
                        feat_hbm.at[sA.at[pl.ds((j + 2) * G, G // 2)]],
                        rows.at[bn].at[pl.ds(0, G // 2)], semg[bn])
                    pltpu.async_copy(
                        feat_hbm.at[sA.at[pl.ds((j + 2) * G + G // 2, G // 2)]],
                        rows.at[bn].at[pl.ds(G // 2, G // 2)], semg[bn])
                else:
                    @pl.when(k < NB // 3 - 1)
                    def _wait_issue():
                        pltpu.make_async_copy(
                            rows.at[bn], h_sh.at[dstc[bn]], sems[bn]).wait()
                        pltpu.async_copy(
                            feat_hbm.at[sA.at[pl.ds((j + 2) * G, G // 2)]],
                            rows.at[bn].at[pl.ds(0, G // 2)], semg[bn])
                        pltpu.async_copy(
                            feat_hbm.at[sA.at[pl.ds((j + 2) * G + G // 2, G // 2)]],
                            rows.at[bn].at[pl.ds(G // 2, G // 2)], semg[bn])

        for b in range(3):
            pltpu.make_async_copy(
                rows.at[b], h_sh.at[dstc[b]], sems[b]).wait()

    plsc.subcore_barrier()

    def _dslice(k):
        return pl.ds(s * RPT + k * RCH, RCH)

    def _hslice(k):
        return pl.ds(c * NP + s * RPT + k * RCH, RCH)

    pltpu.async_copy(h_sh.at[_dslice(0)], rows.at[0], semg[0])
    for k in range(NRCH):
        b = k % 2
        pltpu.make_async_copy(h_sh.at[_dslice(k)], rows.at[b],
                              semg[b]).wait()
        if k + 1 < NRCH:
            if k >= 1:
                pltpu.make_async_copy(rows.at[(k - 1) % 2],
                                      hpart_hbm.at[_hslice(k - 1)],
                                      sems[(k - 1) % 2]).wait()
            pltpu.async_copy(h_sh.at[_dslice(k + 1)], rows.at[(k + 1) % 2],
                             semg[(k + 1) % 2])
        pltpu.async_copy(rows.at[b], hpart_hbm.at[_hslice(k)], sems[b])
    for k in range(NRCH - 2, NRCH):
        pltpu.make_async_copy(rows.at[k % 2], hpart_hbm.at[_hslice(k)],
                              sems[k % 2]).wait()


_sc_scatter = pl.kernel(
    _sc_body,
    out_type=jax.ShapeDtypeStruct((2 * NP, D), jnp.float32),
    mesh=plsc.VectorSubcoreMesh(core_axis_name="c", subcore_axis_name="s"),
    scratch_types=[
        pltpu.VMEM((NTY,), jnp.float32),
        pltpu.VMEM((NBG,), jnp.int32),
        pltpu.VMEM((NBG,), jnp.int32),
        pltpu.VMEM((NBG,), jnp.int32),
        pltpu.VMEM((G,), jnp.int32),
        pltpu.VMEM((G,), jnp.int32),
        pltpu.VMEM((G,), jnp.int32),
        pltpu.VMEM((G,), jnp.float32),
        pltpu.VMEM((G,), jnp.float32),
        pltpu.VMEM((G,), jnp.float32),
        pltpu.VMEM((3, G, D), jnp.float32),
        pltpu.VMEM((NP,), jnp.float32),
        pltpu.VMEM_SHARED((NP, D), jnp.float32),
        pltpu.VMEM_SHARED((NP,), jnp.float32),
        pltpu.SemaphoreType.DMA,
        pltpu.SemaphoreType.DMA,
        pltpu.SemaphoreType.DMA,
        pltpu.SemaphoreType.DMA,
        pltpu.SemaphoreType.DMA,
        pltpu.SemaphoreType.DMA,
    ],
    compiler_params=pltpu.CompilerParams(needs_layout_passes=False,
                                         use_tc_tiling_on_sc=False),
)


BR = 80


def _mm_body(f_ref, h0_ref, h1_ref, w_ref, b_ref, o_ref):
    x = f_ref[...] + h0_ref[...] + h1_ref[...]
    o_ref[...] = (
        jnp.dot(x, w_ref[...], preferred_element_type=jnp.float32) + b_ref[...]
    )


def _tc_matmul(feat, hpart, weight, bias2d):
    return pl.pallas_call(
        _mm_body,
        grid=(N // BR,),
        in_specs=[
            pl.BlockSpec((BR, D), lambda i: (i, 0)),
            pl.BlockSpec((BR, D), lambda i: (i, 0)),
            pl.BlockSpec((BR, D), lambda i: (i + NP // BR, 0)),
            pl.BlockSpec((D, D), lambda i: (0, 0)),
            pl.BlockSpec((1, D), lambda i: (0, 0)),
        ],
        out_specs=pl.BlockSpec((BR, D), lambda i: (i, 0)),
        out_shape=jax.ShapeDtypeStruct((N, D), jnp.float32),
    )(feat, hpart, hpart, weight, bias2d)


def kernel(feat, edge_index, e_feat, weight, bias, edge_weight):
    src = edge_index[0].astype(jnp.int32)
    dst = edge_index[1].astype(jnp.int32)
    ef = e_feat.astype(jnp.int32)
    srcp = jnp.concatenate([src, jnp.asarray(_PAD_SRC)])
    dstp = jnp.concatenate([dst, jnp.asarray(_PAD_DST)])
    efp = jnp.concatenate([ef, jnp.asarray(_PAD_EF)])
    ewt = jax.nn.leaky_relu(edge_weight * ALPHA, negative_slope=0.01)[:, 0]
    ewt2 = jnp.roll(ewt, 1)
    hpart = _sc_scatter(feat, srcp, dstp, efp, ewt2)
    return _tc_matmul(feat, hpart, weight, bias.reshape(1, D))
